# baseline (speedup 1.0000x reference)
import functools

import jax
import jax.numpy as jnp
from jax import lax
from jax.experimental import pallas as pl
from jax.experimental.pallas import tpu as pltpu
from jax.experimental.pallas import tpu_sc as plsc

SEQ = 16384
DIM = 128
NC = 2
NS = 16
NW = NC * NS
B_PER_W = SEQ // NW
CHUNK = 128
NCHUNK = B_PER_W // CHUNK


def _sc_gather_partial_sums(x3, emb):
  mesh = plsc.VectorSubcoreMesh(core_axis_name="c", subcore_axis_name="s")

  nv = DIM // 16

  @functools.partial(
      pl.kernel,
      mesh=mesh,
      out_type=jax.ShapeDtypeStruct((NW, DIM), jnp.float32),
      scratch_types=[
          pltpu.VMEM((NCHUNK, CHUNK), jnp.int32),
          [pltpu.VMEM((CHUNK, DIM), jnp.float32) for _ in range(NCHUNK)],
          pltpu.VMEM((DIM,), jnp.float32),
          [pltpu.SemaphoreType.DMA for _ in range(NCHUNK)],
      ],
  )
  def k(x_hbm, emb_hbm, out_hbm, idx_v, bufs, acc_v, sems):
    wid = lax.axis_index("s") * NC + lax.axis_index("c")
    pltpu.sync_copy(x_hbm.at[wid], idx_v)
    inflight = [
        pltpu.async_copy(emb_hbm.at[idx_v.at[j]], bufs[j], sems[j])
        for j in range(NCHUNK)
    ]

    def accumulate(buf, acc):
      @plsc.parallel_loop(0, CHUNK, unroll=8, carry=acc)
      def final(i, c):
        return tuple(c[d] + buf[i, pl.ds(d * 16, 16)] for d in range(nv))

      return final

    acc = tuple(jnp.zeros((16,), jnp.float32) for _ in range(nv))
    for j in range(NCHUNK):
      inflight[j].wait()
      acc = accumulate(bufs[j], acc)
    for d in range(nv):
      acc_v[pl.ds(d * 16, 16)] = acc[d]
    pltpu.sync_copy(acc_v, out_hbm.at[wid])

  return k(x3, emb)


def _tc_mlp(partials, wh, bh2, wo, bo2):
  def body(p_ref, wh_ref, bh_ref, wo_ref, bo_ref, o_ref):
    embed = jnp.sum(p_ref[...], axis=0, keepdims=True) * (1.0 / SEQ)
    h = jax.lax.dot_general(
        embed, wh_ref[...], (((1,), (1,)), ((), ())),
        preferred_element_type=jnp.float32) + bh_ref[...]
    h = jnp.tanh(h)
    o = jax.lax.dot_general(
        h, wo_ref[...], (((1,), (1,)), ((), ())),
        preferred_element_type=jnp.float32) + bo_ref[...]
    m = jnp.max(o, axis=1, keepdims=True)
    e = jnp.exp(o - m)
    o_ref[...] = e / jnp.sum(e, axis=1, keepdims=True)

  return pl.pallas_call(
      body,
      out_shape=jax.ShapeDtypeStruct((1, 20), jnp.float32),
  )(partials, wh, bh2, wo, bo2)


@jax.jit
def kernel(X, emb, Wh, bh, Wo, bo):
  x3 = X.astype(jnp.int32).reshape(NW, NCHUNK, CHUNK)
  partials = _sc_gather_partial_sums(x3, emb)
  out = _tc_mlp(partials, Wh, bh.reshape(1, -1), Wo, bo.reshape(1, -1))
  return out.reshape(20)


# speedup vs baseline: 1.0240x; 1.0240x over previous
import functools

import jax
import jax.numpy as jnp
from jax import lax
from jax.experimental import pallas as pl
from jax.experimental.pallas import tpu as pltpu
from jax.experimental.pallas import tpu_sc as plsc

SEQ = 16384
DIM = 128
NC = 2
NS = 16
NW = NC * NS
B_PER_W = SEQ // NW
CHUNK = 128
NCHUNK = B_PER_W // CHUNK


def _sc_gather_partial_sums(x3, emb):
  mesh = plsc.VectorSubcoreMesh(core_axis_name="c", subcore_axis_name="s")

  nv = DIM // 16

  @functools.partial(
      pl.kernel,
      mesh=mesh,
      out_type=jax.ShapeDtypeStruct((NW, DIM), jnp.float32),
      scratch_types=[
          pltpu.VMEM((B_PER_W,), jnp.int32),
          pltpu.VMEM((B_PER_W, DIM), jnp.float32),
          pltpu.VMEM((DIM,), jnp.float32),
          pltpu.SemaphoreType.DMA,
      ],
  )
  def k(x_hbm, emb_hbm, out_hbm, idx_v, rows_v, acc_v, sem):
    wid = lax.axis_index("s") * NC + lax.axis_index("c")
    pltpu.sync_copy(x_hbm.at[wid], idx_v)
    cp = pltpu.async_copy(emb_hbm.at[idx_v], rows_v, sem)

    def accumulate(lo, hi, acc):
      @plsc.parallel_loop(lo, hi, carry=acc)
      def final(i, c):
        return tuple(c[d] + rows_v[i, pl.ds(d * 16, 16)] for d in range(nv))

      return final

    acc = tuple(jnp.zeros((16,), jnp.float32) for _ in range(nv))
    cp.wait()
    acc = accumulate(0, B_PER_W, acc)
    for d in range(nv):
      acc_v[pl.ds(d * 16, 16)] = acc[d]
    pltpu.sync_copy(acc_v, out_hbm.at[wid])

  return k(x3, emb)


def _tc_mlp(partials, wh, bh2, wo, bo2):
  def body(p_ref, wh_ref, bh_ref, wo_ref, bo_ref, o_ref):
    embed = jnp.sum(p_ref[...], axis=0, keepdims=True) * (1.0 / SEQ)
    h = jax.lax.dot_general(
        embed, wh_ref[...], (((1,), (1,)), ((), ())),
        preferred_element_type=jnp.float32) + bh_ref[...]
    h = jnp.tanh(h)
    o = jax.lax.dot_general(
        h, wo_ref[...], (((1,), (1,)), ((), ())),
        preferred_element_type=jnp.float32) + bo_ref[...]
    m = jnp.max(o, axis=1, keepdims=True)
    e = jnp.exp(o - m)
    o_ref[...] = e / jnp.sum(e, axis=1, keepdims=True)

  return pl.pallas_call(
      body,
      out_shape=jax.ShapeDtypeStruct((1, 20), jnp.float32),
  )(partials, wh, bh2, wo, bo2)


@jax.jit
def kernel(X, emb, Wh, bh, Wo, bo):
  x3 = X.astype(jnp.int32).reshape(NW, B_PER_W)
  partials = _sc_gather_partial_sums(x3, emb)
  out = _tc_mlp(partials, Wh, bh.reshape(1, -1), Wo, bo.reshape(1, -1))
  return out.reshape(20)
